# baseline (speedup 1.0000x reference)
import jax
import jax.numpy as jnp
from jax import lax
from jax.experimental import pallas as pl
from jax.experimental.pallas import tpu as pltpu
from jax.experimental.pallas import tpu_sc as plsc

N_NODES = 10000
N_EDGES = 320000
N_GRAPHS = 100
D_FEAT = 128
HID = 16
N_MIC = 10
N_ACTS = 8

NC = 2
NS = 16
LANES = 16
NW = NC * NS
CH = 128
ROWS_PER_W = 80
N_ROWS = NW * ROWS_PER_W
E_PAD = N_ROWS * CH
NPAD = 10016
G_PAD = 128

_HIGH = lax.Precision.HIGHEST


def _sc_deg_body(eip, zeros16, ones16, deg_out, idx_v, ones_v, sh_deg, dsem):
    c = lax.axis_index("c")
    s = lax.axis_index("s")

    @pl.when(s == 0)
    def _init():
        pltpu.sync_copy(zeros16, sh_deg)

    pltpu.sync_copy(ones16, ones_v)
    wid = s * NC + c
    pltpu.sync_copy(eip.at[1].at[pl.ds(wid * ROWS_PER_W, ROWS_PER_W)], idx_v)
    plsc.subcore_barrier()

    def group(g, carry):
        descs = [
            pltpu.async_copy(ones_v, sh_deg.at[idx_v.at[8 * g + b]], dsem,
                             add=True)
            for b in range(8)
        ]
        for d in descs:
            d.wait()
        return carry

    lax.fori_loop(0, ROWS_PER_W // 8, group, 0)
    plsc.subcore_barrier()

    @pl.when(s == 0)
    def _flush():
        pltpu.sync_copy(sh_deg, deg_out.at[c])


NBUF = 4


def _sc_agg_body(eip, xn, zeros16, agg_out,
                 sidx, didx, bufs, sh_acc, gsems, ssems):
    c = lax.axis_index("c")
    s = lax.axis_index("s")

    @pl.when(s == 0)
    def _init():
        pltpu.sync_copy(zeros16, sh_acc)

    wid = s * NC + c
    pltpu.sync_copy(eip.at[0].at[pl.ds(wid * ROWS_PER_W, ROWS_PER_W)], sidx)
    pltpu.sync_copy(eip.at[1].at[pl.ds(wid * ROWS_PER_W, ROWS_PER_W)], didx)
    plsc.subcore_barrier()

    for b in range(NBUF):
        pltpu.async_copy(xn.at[sidx.at[b]], bufs[b], gsems[b])

    def step(t, carry):
        j = NBUF * t
        for b in range(NBUF):
            pltpu.make_async_copy(xn.at[sidx.at[j + b]], bufs[b],
                                  gsems[b]).wait()
            pltpu.async_copy(bufs[b], sh_acc.at[didx.at[j + b]], ssems[b],
                             add=True)
        for b in range(NBUF):
            @pl.when(j + b + NBUF < ROWS_PER_W)
            def _next(b=b):
                pltpu.make_async_copy(bufs[b], sh_acc.at[didx.at[j + b]],
                                      ssems[b]).wait()
                pltpu.async_copy(xn.at[sidx.at[j + b + NBUF]], bufs[b],
                                 gsems[b])
        return carry

    lax.fori_loop(0, ROWS_PER_W // NBUF, step, 0)
    for b in range(NBUF):
        pltpu.make_async_copy(bufs[b],
                              sh_acc.at[didx.at[ROWS_PER_W - NBUF + b]],
                              ssems[b]).wait()
    plsc.subcore_barrier()

    @pl.when(s == 0)
    def _flush():
        pltpu.sync_copy(sh_acc, agg_out.at[c])


def _tc_xn_body(x_ref, w_ref, degp_ref, xn_ref):
    deg = degp_ref[0, :N_NODES, :] + degp_ref[1, :N_NODES, :] + 1.0
    dinv = lax.rsqrt(deg)
    xw = jnp.dot(x_ref[...], w_ref[...],
                 preferred_element_type=jnp.float32, precision=_HIGH)
    xn_ref[:N_NODES, :] = xw * dinv
    xn_ref[N_NODES:, :] = jnp.zeros((NPAD - N_NODES, HID), jnp.float32)


def _elu(v):
    return jnp.where(v > 0.0, v, jnp.exp(jnp.minimum(v, 0.0)) - 1.0)


def _tc_head_body(aggp_ref, xn_ref, degp_ref, bconv_ref, batch_ref,
                  w1_ref, b1_ref, w2_ref, b2_ref, wout_ref, bout_ref,
                  out_ref):
    deg = degp_ref[0, :N_NODES, :] + degp_ref[1, :N_NODES, :] + 1.0
    dinv = lax.rsqrt(deg)
    xn = xn_ref[:N_NODES, :]
    agg = aggp_ref[0, :N_NODES, :] + aggp_ref[1, :N_NODES, :]
    h = jnp.maximum(dinv * (agg + xn) + bconv_ref[...], 0.0)

    gid = lax.broadcasted_iota(jnp.int32, (G_PAD, N_NODES), 0)
    ohT = (gid == batch_ref[...]).astype(jnp.float32)
    sums = lax.dot_general(ohT, h, (((1,), (0,)), ((), ())),
                           preferred_element_type=jnp.float32,
                           precision=_HIGH)
    cnt = jnp.sum(ohT, axis=1, keepdims=True)
    g = sums / jnp.maximum(cnt, 1.0)

    g = _elu(jnp.dot(g, w1_ref[...],
                     preferred_element_type=jnp.float32, precision=_HIGH)
             + b1_ref[...])
    g = _elu(jnp.dot(g, w2_ref[...],
                     preferred_element_type=jnp.float32, precision=_HIGH)
             + b2_ref[...])
    out_ref[...] = jnp.dot(g, wout_ref[...],
                           preferred_element_type=jnp.float32,
                           precision=_HIGH) + bout_ref[...]


def kernel(x, edge_index, batch, W_conv, b_conv, W1, b1, W2, b2, W_out, b_out):
    eip = jnp.pad(edge_index.astype(jnp.int32),
                  ((0, 0), (0, E_PAD - N_EDGES)),
                  constant_values=N_NODES).reshape(2, N_ROWS, CH)
    zeros16 = jnp.zeros((NPAD, HID), jnp.float32)
    ones16 = jnp.ones((CH, HID), jnp.float32)

    mesh = plsc.VectorSubcoreMesh(core_axis_name="c", subcore_axis_name="s",
                                  num_cores=NC, num_subcores=NS)
    sc_params = pltpu.CompilerParams(use_tc_tiling_on_sc=False)

    deg_parts = pl.kernel(
        _sc_deg_body,
        out_type=jax.ShapeDtypeStruct((NC, NPAD, HID), jnp.float32),
        mesh=mesh,
        scratch_types=[
            pltpu.VMEM((ROWS_PER_W, CH), jnp.int32),
            pltpu.VMEM((CH, HID), jnp.float32),
            pltpu.VMEM_SHARED((NPAD, HID), jnp.float32),
            pltpu.SemaphoreType.DMA,
        ],
        compiler_params=sc_params,
    )(eip, zeros16, ones16)

    xn = pl.pallas_call(
        _tc_xn_body,
        out_shape=jax.ShapeDtypeStruct((NPAD, HID), jnp.float32),
    )(x, W_conv, deg_parts)

    agg_parts = pl.kernel(
        _sc_agg_body,
        out_type=jax.ShapeDtypeStruct((NC, NPAD, HID), jnp.float32),
        mesh=mesh,
        scratch_types=[
            pltpu.VMEM((ROWS_PER_W, CH), jnp.int32),
            pltpu.VMEM((ROWS_PER_W, CH), jnp.int32),
            [pltpu.VMEM((CH, HID), jnp.float32) for _ in range(NBUF)],
            pltpu.VMEM_SHARED((NPAD, HID), jnp.float32),
            [pltpu.SemaphoreType.DMA for _ in range(NBUF)],
            [pltpu.SemaphoreType.DMA for _ in range(NBUF)],
        ],
        compiler_params=sc_params,
    )(eip, xn, zeros16)

    batch2 = batch.astype(jnp.int32).reshape(1, N_NODES)
    woutr = W_out.transpose(1, 0, 2).reshape(HID * 4, N_MIC * N_ACTS)
    boutr = b_out.reshape(1, N_MIC * N_ACTS)

    outp = pl.pallas_call(
        _tc_head_body,
        out_shape=jax.ShapeDtypeStruct((G_PAD, N_MIC * N_ACTS), jnp.float32),
    )(agg_parts, xn, deg_parts, b_conv.reshape(1, HID), batch2,
      W1, b1.reshape(1, 64), W2, b2.reshape(1, 64), woutr, boutr)

    return outp[:N_GRAPHS].reshape(N_GRAPHS, N_MIC, N_ACTS)
